# baseline (speedup 1.0000x reference)
import functools

import jax
import jax.numpy as jnp
from jax.experimental import pallas as pl
from jax.experimental.pallas import tpu as pltpu

ATOM_DIM = 22
HID = 10
NUM_LAYERS = 7
N = 128
G = 8


def _fused_body(nodes_ref, adj_ref, *refs):
    w_refs = refs[: 3 * NUM_LAYERS]
    Wf1_ref, bf1_ref, Wf2_ref, bf2_ref, out_ref = refs[3 * NUM_LAYERS :]

    A = adj_ref[...]
    rdeg = 1.0 / (jnp.sum(A, axis=2, keepdims=True) + 1e-6)

    x = nodes_ref[...]
    hs = []
    for i in range(NUM_LAYERS):
        Wtop = w_refs[3 * i][...]
        Wbot = w_refs[3 * i + 1][...]
        b = w_refs[3 * i + 2][...]
        d = x.shape[-1]
        xf = x.reshape(G * N, d)
        y = jnp.dot(xf, Wtop, preferred_element_type=jnp.float32)
        z = jnp.dot(xf, Wbot, preferred_element_type=jnp.float32)
        y = y.reshape(G, N, HID)
        z = z.reshape(G, N, HID)
        aggs = [
            jnp.dot(A[g], z[g], preferred_element_type=jnp.float32)[None]
            for g in range(G)
        ]
        agg = jnp.concatenate(aggs, axis=0) * rdeg
        h = jnp.maximum(y + agg + b[None], 0.0)
        if i >= (NUM_LAYERS + 1) // 2:
            h = h + hs[NUM_LAYERS - 1 - i]
        hs.append(h)
        x = h

    mx = jnp.max(x, axis=1)
    mn = jnp.min(x, axis=1)
    sm = jnp.sum(x, axis=1)
    av = sm * (1.0 / N)
    feat = jnp.concatenate([mx, mn, av, sm], axis=1)

    hf = jnp.dot(feat, Wf1_ref[...], preferred_element_type=jnp.float32)
    hf = hf + bf1_ref[...]
    hf = jnp.where(hf > 0, hf, jnp.exp(jnp.minimum(hf, 0.0)) - 1.0)
    out = jnp.dot(hf, Wf2_ref[...], preferred_element_type=jnp.float32)
    out_ref[...] = out + bf2_ref[...]


@functools.partial(jax.jit, static_argnames=())
def kernel(nodes, adj, W0, b0, W1, b1, W2, b2, W3, b3, W4, b4, W5, b5, W6, b6,
           Wf1, bf1, Wf2, bf2):
    B = nodes.shape[0]
    Ws = [W0, W1, W2, W3, W4, W5, W6]
    bs = [b0, b1, b2, b3, b4, b5, b6]
    dims = [ATOM_DIM] + [HID] * NUM_LAYERS

    w_ops = []
    w_specs = []
    for i in range(NUM_LAYERS):
        d = dims[i]
        w_ops += [Ws[i][:d], Ws[i][d:], bs[i].reshape(1, HID)]
        w_specs += [
            pl.BlockSpec((d, HID), lambda i: (0, 0)),
            pl.BlockSpec((d, HID), lambda i: (0, 0)),
            pl.BlockSpec((1, HID), lambda i: (0, 0)),
        ]
    w_ops += [Wf1, bf1.reshape(1, 9), Wf2, bf2.reshape(1, 2)]
    w_specs += [
        pl.BlockSpec((4 * HID, 9), lambda i: (0, 0)),
        pl.BlockSpec((1, 9), lambda i: (0, 0)),
        pl.BlockSpec((9, 2), lambda i: (0, 0)),
        pl.BlockSpec((1, 2), lambda i: (0, 0)),
    ]

    grid = (B // G,)
    return pl.pallas_call(
        _fused_body,
        grid=grid,
        in_specs=[
            pl.BlockSpec((G, N, ATOM_DIM), lambda i: (i, 0, 0)),
            pl.BlockSpec((G, N, N), lambda i: (i, 0, 0)),
            *w_specs,
        ],
        out_specs=pl.BlockSpec((G, 2), lambda i: (i, 0)),
        out_shape=jax.ShapeDtypeStruct((B, 2), jnp.float32),
        compiler_params=pltpu.CompilerParams(
            dimension_semantics=("arbitrary",),
        ),
    )(nodes, adj, *w_ops)


# speedup vs baseline: 1.5590x; 1.5590x over previous
import functools

import jax
import jax.numpy as jnp
from jax import lax
from jax.experimental import pallas as pl
from jax.experimental.pallas import tpu as pltpu

ATOM_DIM = 22
HID = 10
HP = 16
NUM_LAYERS = 7
N = 128
G = 64

_DN_RHS_T = (((1,), (1,)), ((), ()))
_DN_STD = (((1,), (0,)), ((), ()))


def _dot(a, b, dn):
    return lax.dot_general(a, b, dn, preferred_element_type=jnp.float32)


def _fused_body(nodes_ref, adj_ref, *refs):
    wc_refs = refs[:NUM_LAYERS]
    b_refs = refs[NUM_LAYERS : 2 * NUM_LAYERS]
    Wf1_ref, bf1_ref, Wf2_ref, bf2_ref, out_ref = refs[2 * NUM_LAYERS :]

    A = adj_ref[...].astype(jnp.bfloat16)

    ones = jnp.ones((8, N), jnp.bfloat16)
    rdeg = jnp.concatenate(
        [_dot(ones, A[g], _DN_RHS_T)[0:1] for g in range(G)], axis=1
    )
    rdeg = 1.0 / (rdeg + 1e-6)

    xT = None
    hs = []
    for i in range(NUM_LAYERS):
        Wcat = wc_refs[i][...]
        bT = b_refs[i][...]
        if i == 0:
            xf = nodes_ref[...].reshape(G * N, ATOM_DIM).astype(jnp.bfloat16)
            tT = _dot(Wcat, xf, _DN_RHS_T)
        else:
            tT = _dot(Wcat, xT, _DN_STD)
        yT = tT[:HP]
        zT = tT[HP:].astype(jnp.bfloat16)
        aggT = jnp.concatenate(
            [
                _dot(zT[:, g * N : (g + 1) * N], A[g], _DN_RHS_T)
                for g in range(G)
            ],
            axis=1,
        )
        hT = jnp.maximum(yT + aggT * rdeg + bT, 0.0)
        if i >= (NUM_LAYERS + 1) // 2:
            hT = hT + hs[NUM_LAYERS - 1 - i]
        hs.append(hT)
        xT = hT.astype(jnp.bfloat16)

    xr = hs[-1].reshape(HP, G, N)
    mx = jnp.max(xr, axis=2)
    mn = jnp.min(xr, axis=2)
    sm = jnp.sum(xr, axis=2)
    av = sm * (1.0 / N)
    featT = jnp.concatenate([mx, mn, av, sm], axis=0)

    h1 = _dot(Wf1_ref[...], featT, _DN_STD) + bf1_ref[...]
    h1 = jnp.where(h1 > 0, h1, jnp.exp(jnp.minimum(h1, 0.0)) - 1.0)
    outG = lax.dot_general(
        h1, Wf2_ref[...], (((0,), (1,)), ((), ())),
        preferred_element_type=jnp.float32,
    )
    out_ref[...] = outG + bf2_ref[...]


@functools.partial(jax.jit, static_argnames=())
def kernel(nodes, adj, W0, b0, W1, b1, W2, b2, W3, b3, W4, b4, W5, b5, W6, b6,
           Wf1, bf1, Wf2, bf2):
    B = nodes.shape[0]
    Ws = [W0, W1, W2, W3, W4, W5, W6]
    bs = [b0, b1, b2, b3, b4, b5, b6]
    dims = [ATOM_DIM] + [HID] * NUM_LAYERS

    w_ops, w_specs = [], []
    for i in range(NUM_LAYERS):
        d = dims[i]
        dp = d if i == 0 else HP
        pad = jnp.zeros((HP - HID, d), jnp.float32)
        wcat = jnp.concatenate([Ws[i][:d].T, pad, Ws[i][d:].T, pad], axis=0)
        wcat = jnp.pad(wcat, ((0, 0), (0, dp - d)))
        w_ops.append(wcat.astype(jnp.bfloat16))
        w_specs.append(pl.BlockSpec((2 * HP, dp), lambda i: (0, 0)))
    for i in range(NUM_LAYERS):
        bT = jnp.pad(bs[i], (0, HP - HID)).reshape(HP, 1)
        w_ops.append(bT)
        w_specs.append(pl.BlockSpec((HP, 1), lambda i: (0, 0)))

    wf1e = jnp.pad(
        Wf1.reshape(4, HID, 9), ((0, 0), (0, HP - HID), (0, HP - 9))
    )
    wf1e = wf1e.transpose(2, 0, 1).reshape(HP, 4 * HP)
    bf1e = jnp.pad(bf1, (0, HP - 9)).reshape(HP, 1)
    wf2e = jnp.pad(Wf2.T, ((0, 6), (0, HP - 9)))
    bf2e = jnp.pad(bf2, (0, 6)).reshape(1, 8)
    w_ops += [wf1e, bf1e, wf2e, bf2e]
    w_specs += [
        pl.BlockSpec((HP, 4 * HP), lambda i: (0, 0)),
        pl.BlockSpec((HP, 1), lambda i: (0, 0)),
        pl.BlockSpec((8, HP), lambda i: (0, 0)),
        pl.BlockSpec((1, 8), lambda i: (0, 0)),
    ]

    grid = (B // G,)
    outG = pl.pallas_call(
        _fused_body,
        grid=grid,
        in_specs=[
            pl.BlockSpec((G, N, ATOM_DIM), lambda i: (i, 0, 0)),
            pl.BlockSpec((G, N, N), lambda i: (i, 0, 0)),
            *w_specs,
        ],
        out_specs=pl.BlockSpec((G, 8), lambda i: (i, 0)),
        out_shape=jax.ShapeDtypeStruct((B, 8), jnp.float32),
        compiler_params=pltpu.CompilerParams(
            dimension_semantics=("arbitrary",),
        ),
    )(nodes, adj, *w_ops)
    return outG[:, :2]


# speedup vs baseline: 1.8096x; 1.1608x over previous
import functools

import jax
import jax.numpy as jnp
from jax import lax
from jax.experimental import pallas as pl
from jax.experimental.pallas import tpu as pltpu

ATOM_DIM = 22
HID = 10
HP = 16
NUM_LAYERS = 7
N = 128
G = 64

_DN_RHS_T = (((1,), (1,)), ((), ()))
_DN_STD = (((1,), (0,)), ((), ()))


def _dot(a, b, dn):
    return lax.dot_general(a, b, dn, preferred_element_type=jnp.float32)


def _fused_body(nodes_ref, adj_ref, *refs):
    wc_refs = refs[:NUM_LAYERS]
    b_refs = refs[NUM_LAYERS : 2 * NUM_LAYERS]
    Wf1_ref, bf1_ref, Wf2_ref, bf2_ref, out_ref = refs[2 * NUM_LAYERS :]

    A = adj_ref[...].astype(jnp.bfloat16)

    ones = jnp.ones((8, N), jnp.bfloat16)
    rdeg = jnp.concatenate(
        [_dot(ones, A[g], _DN_RHS_T)[0:1] for g in range(G)], axis=1
    )
    rdeg = 1.0 / (rdeg + 1e-6)

    xT = None
    hs = []
    for i in range(NUM_LAYERS):
        Wcat = wc_refs[i][...]
        bT = b_refs[i][...]
        if i == 0:
            tT = _dot(Wcat, nodes_ref[...].astype(jnp.bfloat16), _DN_STD)
        else:
            tT = _dot(Wcat, xT, _DN_STD)
        yT = tT[:HP]
        zT = tT[HP:].astype(jnp.bfloat16)
        aggT = jnp.concatenate(
            [
                _dot(zT[:, g * N : (g + 1) * N], A[g], _DN_RHS_T)
                for g in range(G)
            ],
            axis=1,
        )
        hT = jnp.maximum(yT + aggT * rdeg + bT, 0.0)
        if i >= (NUM_LAYERS + 1) // 2:
            hT = hT + hs[NUM_LAYERS - 1 - i]
        hs.append(hT)
        xT = hT.astype(jnp.bfloat16)

    xr = hs[-1].reshape(HP, G, N)
    mx = jnp.max(xr, axis=2)
    mn = jnp.min(xr, axis=2)
    sm = jnp.sum(xr, axis=2)
    av = sm * (1.0 / N)
    featT = jnp.concatenate([mx, mn, av, sm], axis=0)

    h1 = _dot(Wf1_ref[...], featT, _DN_STD) + bf1_ref[...]
    h1 = jnp.where(h1 > 0, h1, jnp.exp(jnp.minimum(h1, 0.0)) - 1.0)
    outG = lax.dot_general(
        h1, Wf2_ref[...], (((0,), (1,)), ((), ())),
        preferred_element_type=jnp.float32,
    )
    out_ref[...] = outG + bf2_ref[...]


@functools.partial(jax.jit, static_argnames=())
def kernel(nodes, adj, W0, b0, W1, b1, W2, b2, W3, b3, W4, b4, W5, b5, W6, b6,
           Wf1, bf1, Wf2, bf2):
    B = nodes.shape[0]
    Ws = [W0, W1, W2, W3, W4, W5, W6]
    bs = [b0, b1, b2, b3, b4, b5, b6]
    dims = [ATOM_DIM] + [HID] * NUM_LAYERS

    w_ops, w_specs = [], []
    for i in range(NUM_LAYERS):
        d = dims[i]
        dp = d if i == 0 else HP
        pad = jnp.zeros((HP - HID, d), jnp.float32)
        wcat = jnp.concatenate([Ws[i][:d].T, pad, Ws[i][d:].T, pad], axis=0)
        wcat = jnp.pad(wcat, ((0, 0), (0, dp - d)))
        w_ops.append(wcat.astype(jnp.bfloat16))
        w_specs.append(pl.BlockSpec((2 * HP, dp), lambda i: (0, 0)))
    for i in range(NUM_LAYERS):
        bT = jnp.pad(bs[i], (0, HP - HID)).reshape(HP, 1)
        w_ops.append(bT)
        w_specs.append(pl.BlockSpec((HP, 1), lambda i: (0, 0)))

    wf1e = jnp.pad(
        Wf1.reshape(4, HID, 9), ((0, 0), (0, HP - HID), (0, HP - 9))
    )
    wf1e = wf1e.transpose(2, 0, 1).reshape(HP, 4 * HP)
    bf1e = jnp.pad(bf1, (0, HP - 9)).reshape(HP, 1)
    wf2e = jnp.pad(Wf2.T, ((0, 6), (0, HP - 9)))
    bf2e = jnp.pad(bf2, (0, 6)).reshape(1, 8)
    w_ops += [wf1e, bf1e, wf2e, bf2e]
    w_specs += [
        pl.BlockSpec((HP, 4 * HP), lambda i: (0, 0)),
        pl.BlockSpec((HP, 1), lambda i: (0, 0)),
        pl.BlockSpec((8, HP), lambda i: (0, 0)),
        pl.BlockSpec((1, 8), lambda i: (0, 0)),
    ]

    nodesT = jnp.transpose(nodes, (2, 0, 1)).reshape(ATOM_DIM, B * N)

    grid = (B // G,)
    outG = pl.pallas_call(
        _fused_body,
        grid=grid,
        in_specs=[
            pl.BlockSpec((ATOM_DIM, G * N), lambda i: (0, i)),
            pl.BlockSpec((G, N, N), lambda i: (i, 0, 0)),
            *w_specs,
        ],
        out_specs=pl.BlockSpec((G, 8), lambda i: (i, 0)),
        out_shape=jax.ShapeDtypeStruct((B, 8), jnp.float32),
        compiler_params=pltpu.CompilerParams(
            dimension_semantics=("arbitrary",),
        ),
    )(nodesT, adj, *w_ops)
    return outG[:, :2]


# speedup vs baseline: 1.8105x; 1.0005x over previous
import functools

import jax
import jax.numpy as jnp
from jax import lax
from jax.experimental import pallas as pl
from jax.experimental.pallas import tpu as pltpu

ATOM_DIM = 22
HID = 10
HP = 16
NUM_LAYERS = 7
N = 128
G = 64

_DN_RHS_T = (((1,), (1,)), ((), ()))
_DN_STD = (((1,), (0,)), ((), ()))


def _dot(a, b, dn):
    return lax.dot_general(a, b, dn, preferred_element_type=jnp.float32)


def _fused_body(nodes_ref, adj_ref, *refs):
    wc_refs = refs[:NUM_LAYERS]
    b_refs = refs[NUM_LAYERS : 2 * NUM_LAYERS]
    Wf1_ref, bf1_ref, Wf2_ref, bf2_ref, out_ref = refs[2 * NUM_LAYERS :]

    A = adj_ref[...].astype(jnp.bfloat16)

    ones = jnp.ones((8, N), jnp.bfloat16)
    rdeg = jnp.concatenate(
        [_dot(ones, A[g], _DN_RHS_T)[0:1] for g in range(G)], axis=1
    )
    rdeg = 1.0 / (rdeg + 1e-6)

    xT = None
    hs = []
    for i in range(NUM_LAYERS):
        Wcat = wc_refs[i][...]
        bT = b_refs[i][...]
        if i == 0:
            tT = _dot(Wcat, nodes_ref[...].astype(jnp.bfloat16), _DN_STD)
        else:
            tT = _dot(Wcat, xT, _DN_STD)
        yT = tT[:HP]
        zT = tT[HP:].astype(jnp.bfloat16)
        aggT = jnp.concatenate(
            [
                _dot(zT[:, g * N : (g + 1) * N], A[g], _DN_RHS_T)
                for g in range(G)
            ],
            axis=1,
        )
        hT = jnp.maximum(yT + aggT * rdeg + bT, 0.0)
        if i >= (NUM_LAYERS + 1) // 2:
            hT = hT + hs[NUM_LAYERS - 1 - i]
        hs.append(hT)
        xT = hT.astype(jnp.bfloat16)

    xr = hs[-1].reshape(HP, G, N)
    mx = jnp.max(xr, axis=2)
    mn = jnp.min(xr, axis=2)
    sm = jnp.sum(xr, axis=2)
    av = sm * (1.0 / N)
    featT = jnp.concatenate([mx, mn, av, sm], axis=0)

    h1 = _dot(Wf1_ref[...], featT, _DN_STD) + bf1_ref[...]
    h1 = jnp.where(h1 > 0, h1, jnp.exp(jnp.minimum(h1, 0.0)) - 1.0)
    outG = lax.dot_general(
        h1, Wf2_ref[...], (((0,), (1,)), ((), ())),
        preferred_element_type=jnp.float32,
    )
    out_ref[...] = outG + bf2_ref[...]


@functools.partial(jax.jit, static_argnames=())
def kernel(nodes, adj, W0, b0, W1, b1, W2, b2, W3, b3, W4, b4, W5, b5, W6, b6,
           Wf1, bf1, Wf2, bf2):
    B = nodes.shape[0]
    Ws = [W0, W1, W2, W3, W4, W5, W6]
    bs = [b0, b1, b2, b3, b4, b5, b6]
    dims = [ATOM_DIM] + [HID] * NUM_LAYERS

    w_ops, w_specs = [], []
    for i in range(NUM_LAYERS):
        d = dims[i]
        dp = d if i == 0 else HP
        pad = jnp.zeros((HP - HID, d), jnp.float32)
        wcat = jnp.concatenate([Ws[i][:d].T, pad, Ws[i][d:].T, pad], axis=0)
        wcat = jnp.pad(wcat, ((0, 0), (0, dp - d)))
        w_ops.append(wcat.astype(jnp.bfloat16))
        w_specs.append(pl.BlockSpec((2 * HP, dp), lambda i: (0, 0)))
    for i in range(NUM_LAYERS):
        bT = jnp.pad(bs[i], (0, HP - HID)).reshape(HP, 1)
        w_ops.append(bT)
        w_specs.append(pl.BlockSpec((HP, 1), lambda i: (0, 0)))

    wf1e = jnp.pad(
        Wf1.reshape(4, HID, 9), ((0, 0), (0, HP - HID), (0, HP - 9))
    )
    wf1e = wf1e.transpose(2, 0, 1).reshape(HP, 4 * HP)
    bf1e = jnp.pad(bf1, (0, HP - 9)).reshape(HP, 1)
    wf2e = jnp.pad(Wf2.T, ((0, 6), (0, HP - 9)))
    bf2e = jnp.pad(bf2, (0, 6)).reshape(1, 8)
    w_ops += [wf1e, bf1e, wf2e, bf2e]
    w_specs += [
        pl.BlockSpec((HP, 4 * HP), lambda i: (0, 0)),
        pl.BlockSpec((HP, 1), lambda i: (0, 0)),
        pl.BlockSpec((8, HP), lambda i: (0, 0)),
        pl.BlockSpec((1, 8), lambda i: (0, 0)),
    ]

    nodesT = jnp.transpose(nodes, (2, 0, 1)).reshape(ATOM_DIM, B * N)

    grid = (B // G,)
    outG = pl.pallas_call(
        _fused_body,
        grid=grid,
        in_specs=[
            pl.BlockSpec((ATOM_DIM, G * N), lambda i: (0, i)),
            pl.BlockSpec((G, N, N), lambda i: (i, 0, 0)),
            *w_specs,
        ],
        out_specs=pl.BlockSpec((G, 8), lambda i: (i, 0)),
        out_shape=jax.ShapeDtypeStruct((B, 8), jnp.float32),
        compiler_params=pltpu.CompilerParams(
            dimension_semantics=("parallel",),
        ),
    )(nodesT, adj, *w_ops)
    return outG[:, :2]


# speedup vs baseline: 1.8581x; 1.0262x over previous
import functools

import jax
import jax.numpy as jnp
from jax import lax
from jax.experimental import pallas as pl
from jax.experimental.pallas import tpu as pltpu

ATOM_DIM = 22
HID = 10
HP = 16
NUM_LAYERS = 7
N = 128
G = 128

_DN_RHS_T = (((1,), (1,)), ((), ()))
_DN_STD = (((1,), (0,)), ((), ()))


def _dot(a, b, dn):
    return lax.dot_general(a, b, dn, preferred_element_type=jnp.float32)


def _fused_body(nodes_ref, adj_ref, *refs):
    wc_refs = refs[:NUM_LAYERS]
    b_refs = refs[NUM_LAYERS : 2 * NUM_LAYERS]
    Wf1_ref, bf1_ref, Wf2_ref, bf2_ref, out_ref = refs[2 * NUM_LAYERS :]

    A = adj_ref[...].astype(jnp.bfloat16)

    ones = jnp.ones((8, N), jnp.bfloat16)
    rdeg = jnp.concatenate(
        [_dot(ones, A[g], _DN_RHS_T)[0:1] for g in range(G)], axis=1
    )
    rdeg = 1.0 / (rdeg + 1e-6)

    xT = None
    hs = []
    for i in range(NUM_LAYERS):
        Wcat = wc_refs[i][...]
        bT = b_refs[i][...]
        if i == 0:
            tT = _dot(Wcat, nodes_ref[...].astype(jnp.bfloat16), _DN_STD)
        else:
            tT = _dot(Wcat, xT, _DN_STD)
        yT = tT[:HP]
        zT = tT[HP:].astype(jnp.bfloat16)
        aggT = jnp.concatenate(
            [
                _dot(zT[:, g * N : (g + 1) * N], A[g], _DN_RHS_T)
                for g in range(G)
            ],
            axis=1,
        )
        hT = jnp.maximum(yT + aggT * rdeg + bT, 0.0)
        if i >= (NUM_LAYERS + 1) // 2:
            hT = hT + hs[NUM_LAYERS - 1 - i]
        hs.append(hT)
        xT = hT.astype(jnp.bfloat16)

    xr = hs[-1].reshape(HP, G, N)
    mx = jnp.max(xr, axis=2)
    mn = jnp.min(xr, axis=2)
    sm = jnp.sum(xr, axis=2)
    av = sm * (1.0 / N)
    featT = jnp.concatenate([mx, mn, av, sm], axis=0)

    h1 = _dot(Wf1_ref[...], featT, _DN_STD) + bf1_ref[...]
    h1 = jnp.where(h1 > 0, h1, jnp.exp(jnp.minimum(h1, 0.0)) - 1.0)
    outG = lax.dot_general(
        h1, Wf2_ref[...], (((0,), (1,)), ((), ())),
        preferred_element_type=jnp.float32,
    )
    out_ref[...] = outG + bf2_ref[...]


@functools.partial(jax.jit, static_argnames=())
def kernel(nodes, adj, W0, b0, W1, b1, W2, b2, W3, b3, W4, b4, W5, b5, W6, b6,
           Wf1, bf1, Wf2, bf2):
    B = nodes.shape[0]
    Ws = [W0, W1, W2, W3, W4, W5, W6]
    bs = [b0, b1, b2, b3, b4, b5, b6]
    dims = [ATOM_DIM] + [HID] * NUM_LAYERS

    w_ops, w_specs = [], []
    for i in range(NUM_LAYERS):
        d = dims[i]
        dp = d if i == 0 else HP
        pad = jnp.zeros((HP - HID, d), jnp.float32)
        wcat = jnp.concatenate([Ws[i][:d].T, pad, Ws[i][d:].T, pad], axis=0)
        wcat = jnp.pad(wcat, ((0, 0), (0, dp - d)))
        w_ops.append(wcat.astype(jnp.bfloat16))
        w_specs.append(pl.BlockSpec((2 * HP, dp), lambda i: (0, 0)))
    for i in range(NUM_LAYERS):
        bT = jnp.pad(bs[i], (0, HP - HID)).reshape(HP, 1)
        w_ops.append(bT)
        w_specs.append(pl.BlockSpec((HP, 1), lambda i: (0, 0)))

    wf1e = jnp.pad(
        Wf1.reshape(4, HID, 9), ((0, 0), (0, HP - HID), (0, HP - 9))
    )
    wf1e = wf1e.transpose(2, 0, 1).reshape(HP, 4 * HP)
    bf1e = jnp.pad(bf1, (0, HP - 9)).reshape(HP, 1)
    wf2e = jnp.pad(Wf2.T, ((0, 6), (0, HP - 9)))
    bf2e = jnp.pad(bf2, (0, 6)).reshape(1, 8)
    w_ops += [wf1e, bf1e, wf2e, bf2e]
    w_specs += [
        pl.BlockSpec((HP, 4 * HP), lambda i: (0, 0)),
        pl.BlockSpec((HP, 1), lambda i: (0, 0)),
        pl.BlockSpec((8, HP), lambda i: (0, 0)),
        pl.BlockSpec((1, 8), lambda i: (0, 0)),
    ]

    nodesT = jnp.transpose(nodes, (2, 0, 1)).reshape(ATOM_DIM, B * N)

    grid = (B // G,)
    outG = pl.pallas_call(
        _fused_body,
        grid=grid,
        in_specs=[
            pl.BlockSpec((ATOM_DIM, G * N), lambda i: (0, i)),
            pl.BlockSpec((G, N, N), lambda i: (i, 0, 0)),
            *w_specs,
        ],
        out_specs=pl.BlockSpec((G, 8), lambda i: (i, 0)),
        out_shape=jax.ShapeDtypeStruct((B, 8), jnp.float32),
        compiler_params=pltpu.CompilerParams(
            dimension_semantics=("parallel",),
        ),
    )(nodesT, adj, *w_ops)
    return outG[:, :2]


# speedup vs baseline: 2.9251x; 1.5743x over previous
import functools

import jax
import jax.numpy as jnp
from jax import lax
from jax.experimental import pallas as pl
from jax.experimental.pallas import tpu as pltpu

ATOM_DIM = 22
HID = 10
HP = 16
NUM_LAYERS = 7
N = 128
G = 128

_DN_RHS_T = (((1,), (1,)), ((), ()))
_DN_STD = (((1,), (0,)), ((), ()))


def _dot(a, b, dn):
    return lax.dot_general(a, b, dn, preferred_element_type=jnp.float32)


def _fused_body(nodes_ref, adj_ref, *refs):
    wc_refs = refs[:NUM_LAYERS]
    b_refs = refs[NUM_LAYERS : 2 * NUM_LAYERS]
    Wf1_ref, bf1_ref, Wf2_ref, bf2_ref, out_ref = refs[2 * NUM_LAYERS :]

    A = adj_ref[...]
    AT = jnp.transpose(A, (0, 2, 1))

    ones = jnp.ones((8, N), jnp.float32)
    rdeg = jnp.concatenate(
        [_dot(ones, AT[g], _DN_STD)[0:1] for g in range(G)], axis=1
    )
    rdeg = 1.0 / (rdeg + 1e-6)

    xT = None
    hs = []
    for i in range(NUM_LAYERS):
        Wcat = wc_refs[i][...]
        bT = b_refs[i][...]
        if i == 0:
            tT = _dot(Wcat, nodes_ref[...], _DN_STD)
        else:
            tT = _dot(Wcat, xT, _DN_STD)
        yT = tT[:HP]
        zT = tT[HP:]
        aggT = jnp.concatenate(
            [
                _dot(zT[:, g * N : (g + 1) * N], AT[g], _DN_STD)
                for g in range(G)
            ],
            axis=1,
        )
        hT = jnp.maximum(yT + aggT * rdeg + bT, 0.0)
        if i >= (NUM_LAYERS + 1) // 2:
            hT = hT + hs[NUM_LAYERS - 1 - i]
        hs.append(hT)
        xT = hT

    xr = hs[-1].reshape(HP, G, N)
    mx = jnp.max(xr, axis=2)
    mn = jnp.min(xr, axis=2)
    sm = jnp.sum(xr, axis=2)
    av = sm * (1.0 / N)
    featT = jnp.concatenate([mx, mn, av, sm], axis=0)

    h1 = _dot(Wf1_ref[...], featT, _DN_STD) + bf1_ref[...]
    h1 = jnp.where(h1 > 0, h1, jnp.exp(jnp.minimum(h1, 0.0)) - 1.0)
    outG = lax.dot_general(
        h1, Wf2_ref[...], (((0,), (1,)), ((), ())),
        preferred_element_type=jnp.float32,
    )
    out_ref[...] = outG + bf2_ref[...]


@functools.partial(jax.jit, static_argnames=())
def kernel(nodes, adj, W0, b0, W1, b1, W2, b2, W3, b3, W4, b4, W5, b5, W6, b6,
           Wf1, bf1, Wf2, bf2):
    B = nodes.shape[0]
    Ws = [W0, W1, W2, W3, W4, W5, W6]
    bs = [b0, b1, b2, b3, b4, b5, b6]
    dims = [ATOM_DIM] + [HID] * NUM_LAYERS

    w_ops, w_specs = [], []
    for i in range(NUM_LAYERS):
        d = dims[i]
        dp = d if i == 0 else HP
        pad = jnp.zeros((HP - HID, d), jnp.float32)
        wcat = jnp.concatenate([Ws[i][:d].T, pad, Ws[i][d:].T, pad], axis=0)
        wcat = jnp.pad(wcat, ((0, 0), (0, dp - d)))
        w_ops.append(wcat)
        w_specs.append(pl.BlockSpec((2 * HP, dp), lambda i: (0, 0)))
    for i in range(NUM_LAYERS):
        bT = jnp.pad(bs[i], (0, HP - HID)).reshape(HP, 1)
        w_ops.append(bT)
        w_specs.append(pl.BlockSpec((HP, 1), lambda i: (0, 0)))

    wf1e = jnp.pad(
        Wf1.reshape(4, HID, 9), ((0, 0), (0, HP - HID), (0, HP - 9))
    )
    wf1e = wf1e.transpose(2, 0, 1).reshape(HP, 4 * HP)
    bf1e = jnp.pad(bf1, (0, HP - 9)).reshape(HP, 1)
    wf2e = jnp.pad(Wf2.T, ((0, 6), (0, HP - 9)))
    bf2e = jnp.pad(bf2, (0, 6)).reshape(1, 8)
    w_ops += [wf1e, bf1e, wf2e, bf2e]
    w_specs += [
        pl.BlockSpec((HP, 4 * HP), lambda i: (0, 0)),
        pl.BlockSpec((HP, 1), lambda i: (0, 0)),
        pl.BlockSpec((8, HP), lambda i: (0, 0)),
        pl.BlockSpec((1, 8), lambda i: (0, 0)),
    ]

    nodesT = jnp.transpose(nodes, (2, 0, 1)).reshape(ATOM_DIM, B * N)

    grid = (B // G,)
    outG = pl.pallas_call(
        _fused_body,
        grid=grid,
        in_specs=[
            pl.BlockSpec((ATOM_DIM, G * N), lambda i: (0, i)),
            pl.BlockSpec((G, N, N), lambda i: (i, 0, 0)),
            *w_specs,
        ],
        out_specs=pl.BlockSpec((G, 8), lambda i: (i, 0)),
        out_shape=jax.ShapeDtypeStruct((B, 8), jnp.float32),
        compiler_params=pltpu.CompilerParams(
            dimension_semantics=("parallel",),
        ),
    )(nodesT, adj, *w_ops)
    return outG[:, :2]


# speedup vs baseline: 3.3253x; 1.1368x over previous
import functools

import jax
import jax.numpy as jnp
from jax import lax
from jax.experimental import pallas as pl
from jax.experimental.pallas import tpu as pltpu

ATOM_DIM = 22
HID = 10
HP = 16
NUM_LAYERS = 7
N = 128
G = 128

_DN_RHS_T = (((1,), (1,)), ((), ()))
_DN_STD = (((1,), (0,)), ((), ()))


def _dot(a, b, dn):
    return lax.dot_general(a, b, dn, preferred_element_type=jnp.float32)


def _fused_body(nodes_ref, adj_ref, *refs):
    wc_refs = refs[:NUM_LAYERS]
    b_refs = refs[NUM_LAYERS : 2 * NUM_LAYERS]
    Wf1_ref, bf1_ref, Wf2_ref, bf2_ref, out_ref = refs[2 * NUM_LAYERS :]

    A = adj_ref[...]
    AT = jnp.transpose(A, (0, 2, 1))

    ones = jnp.ones((8, N), jnp.float32)
    rdeg = jnp.concatenate(
        [_dot(ones, AT[g], _DN_STD)[0:1] for g in range(G)], axis=1
    )
    rdeg = 1.0 / (rdeg + 1e-6)

    xT = None
    hs = []
    for i in range(NUM_LAYERS):
        Wcat = wc_refs[i][...]
        bT = b_refs[i][...]
        if i == 0:
            tT = _dot(Wcat, nodes_ref[...].reshape(ATOM_DIM, G * N), _DN_STD)
        else:
            tT = _dot(Wcat, xT, _DN_STD)
        yT = tT[:HP]
        zT = tT[HP:]
        aggT = jnp.concatenate(
            [
                _dot(zT[:, g * N : (g + 1) * N], AT[g], _DN_STD)
                for g in range(G)
            ],
            axis=1,
        )
        hT = jnp.maximum(yT + aggT * rdeg + bT, 0.0)
        if i >= (NUM_LAYERS + 1) // 2:
            hT = hT + hs[NUM_LAYERS - 1 - i]
        hs.append(hT)
        xT = hT

    xr = hs[-1].reshape(HP, G, N)
    mx = jnp.max(xr, axis=2)
    mn = jnp.min(xr, axis=2)
    sm = jnp.sum(xr, axis=2)
    av = sm * (1.0 / N)
    featT = jnp.concatenate([mx, mn, av, sm], axis=0)

    h1 = _dot(Wf1_ref[...], featT, _DN_STD) + bf1_ref[...]
    h1 = jnp.where(h1 > 0, h1, jnp.exp(jnp.minimum(h1, 0.0)) - 1.0)
    outG = lax.dot_general(
        h1, Wf2_ref[...], (((0,), (1,)), ((), ())),
        preferred_element_type=jnp.float32,
    )
    out_ref[...] = outG + bf2_ref[...]


@functools.partial(jax.jit, static_argnames=())
def kernel(nodes, adj, W0, b0, W1, b1, W2, b2, W3, b3, W4, b4, W5, b5, W6, b6,
           Wf1, bf1, Wf2, bf2):
    B = nodes.shape[0]
    Ws = [W0, W1, W2, W3, W4, W5, W6]
    bs = [b0, b1, b2, b3, b4, b5, b6]
    dims = [ATOM_DIM] + [HID] * NUM_LAYERS

    w_ops, w_specs = [], []
    for i in range(NUM_LAYERS):
        d = dims[i]
        dp = d if i == 0 else HP
        pad = jnp.zeros((HP - HID, d), jnp.float32)
        wcat = jnp.concatenate([Ws[i][:d].T, pad, Ws[i][d:].T, pad], axis=0)
        wcat = jnp.pad(wcat, ((0, 0), (0, dp - d)))
        w_ops.append(wcat)
        w_specs.append(pl.BlockSpec((2 * HP, dp), lambda i: (0, 0)))
    for i in range(NUM_LAYERS):
        bT = jnp.pad(bs[i], (0, HP - HID)).reshape(HP, 1)
        w_ops.append(bT)
        w_specs.append(pl.BlockSpec((HP, 1), lambda i: (0, 0)))

    wf1e = jnp.pad(
        Wf1.reshape(4, HID, 9), ((0, 0), (0, HP - HID), (0, HP - 9))
    )
    wf1e = wf1e.transpose(2, 0, 1).reshape(HP, 4 * HP)
    bf1e = jnp.pad(bf1, (0, HP - 9)).reshape(HP, 1)
    wf2e = jnp.pad(Wf2.T, ((0, 6), (0, HP - 9)))
    bf2e = jnp.pad(bf2, (0, 6)).reshape(1, 8)
    w_ops += [wf1e, bf1e, wf2e, bf2e]
    w_specs += [
        pl.BlockSpec((HP, 4 * HP), lambda i: (0, 0)),
        pl.BlockSpec((HP, 1), lambda i: (0, 0)),
        pl.BlockSpec((8, HP), lambda i: (0, 0)),
        pl.BlockSpec((1, 8), lambda i: (0, 0)),
    ]

    nodesT = jnp.transpose(nodes, (2, 0, 1))

    grid = (B // G,)
    outG = pl.pallas_call(
        _fused_body,
        grid=grid,
        in_specs=[
            pl.BlockSpec((ATOM_DIM, G, N), lambda i: (0, i, 0)),
            pl.BlockSpec((G, N, N), lambda i: (i, 0, 0)),
            *w_specs,
        ],
        out_specs=pl.BlockSpec((G, 8), lambda i: (i, 0)),
        out_shape=jax.ShapeDtypeStruct((B, 8), jnp.float32),
        compiler_params=pltpu.CompilerParams(
            dimension_semantics=("parallel",),
        ),
    )(nodesT, adj, *w_ops)
    return outG[:, :2]


# speedup vs baseline: 3.3559x; 1.0092x over previous
import functools

import jax
import jax.numpy as jnp
from jax import lax
from jax.experimental import pallas as pl
from jax.experimental.pallas import tpu as pltpu

ATOM_DIM = 22
HID = 10
HP = 16
NUM_LAYERS = 7
N = 128
G = 128

_DN_RHS_T = (((1,), (1,)), ((), ()))
_DN_STD = (((1,), (0,)), ((), ()))


def _dot(a, b, dn):
    return lax.dot_general(a, b, dn, preferred_element_type=jnp.float32)


def _fused_body(nodes_ref, adj_ref, *refs):
    wc_refs = refs[:NUM_LAYERS]
    b_refs = refs[NUM_LAYERS : 2 * NUM_LAYERS]
    Wf1_ref, bf1_ref, Wf2_ref, bf2_ref, out_ref = refs[2 * NUM_LAYERS :]

    A = adj_ref[...]
    AT = jnp.transpose(A, (0, 2, 1))

    rdeg = jnp.sum(AT, axis=1).reshape(1, G * N)
    rdeg = 1.0 / (rdeg + 1e-6)

    xT = None
    hs = []
    for i in range(NUM_LAYERS):
        Wcat = wc_refs[i][...]
        bT = b_refs[i][...]
        if i == 0:
            tT = _dot(Wcat, nodes_ref[...].reshape(ATOM_DIM, G * N), _DN_STD)
        else:
            tT = _dot(Wcat, xT, _DN_STD)
        yT = tT[:HP]
        zT = tT[HP:]
        aggT = jnp.concatenate(
            [
                _dot(zT[:, g * N : (g + 1) * N], AT[g], _DN_STD)
                for g in range(G)
            ],
            axis=1,
        )
        hT = jnp.maximum(yT + aggT * rdeg + bT, 0.0)
        if i >= (NUM_LAYERS + 1) // 2:
            hT = hT + hs[NUM_LAYERS - 1 - i]
        hs.append(hT)
        xT = hT

    xr = hs[-1].reshape(HP, G, N)
    mx = jnp.max(xr, axis=2)
    mn = jnp.min(xr, axis=2)
    sm = jnp.sum(xr, axis=2)
    av = sm * (1.0 / N)
    featT = jnp.concatenate([mx, mn, av, sm], axis=0)

    h1 = _dot(Wf1_ref[...], featT, _DN_STD) + bf1_ref[...]
    h1 = jnp.where(h1 > 0, h1, jnp.exp(jnp.minimum(h1, 0.0)) - 1.0)
    outG = lax.dot_general(
        h1, Wf2_ref[...], (((0,), (1,)), ((), ())),
        preferred_element_type=jnp.float32,
    )
    out_ref[...] = outG + bf2_ref[...]


@functools.partial(jax.jit, static_argnames=())
def kernel(nodes, adj, W0, b0, W1, b1, W2, b2, W3, b3, W4, b4, W5, b5, W6, b6,
           Wf1, bf1, Wf2, bf2):
    B = nodes.shape[0]
    Ws = [W0, W1, W2, W3, W4, W5, W6]
    bs = [b0, b1, b2, b3, b4, b5, b6]
    dims = [ATOM_DIM] + [HID] * NUM_LAYERS

    w_ops, w_specs = [], []
    for i in range(NUM_LAYERS):
        d = dims[i]
        dp = d if i == 0 else HP
        pad = jnp.zeros((HP - HID, d), jnp.float32)
        wcat = jnp.concatenate([Ws[i][:d].T, pad, Ws[i][d:].T, pad], axis=0)
        wcat = jnp.pad(wcat, ((0, 0), (0, dp - d)))
        w_ops.append(wcat)
        w_specs.append(pl.BlockSpec((2 * HP, dp), lambda i: (0, 0)))
    for i in range(NUM_LAYERS):
        bT = jnp.pad(bs[i], (0, HP - HID)).reshape(HP, 1)
        w_ops.append(bT)
        w_specs.append(pl.BlockSpec((HP, 1), lambda i: (0, 0)))

    wf1e = jnp.pad(
        Wf1.reshape(4, HID, 9), ((0, 0), (0, HP - HID), (0, HP - 9))
    )
    wf1e = wf1e.transpose(2, 0, 1).reshape(HP, 4 * HP)
    bf1e = jnp.pad(bf1, (0, HP - 9)).reshape(HP, 1)
    wf2e = jnp.pad(Wf2.T, ((0, 6), (0, HP - 9)))
    bf2e = jnp.pad(bf2, (0, 6)).reshape(1, 8)
    w_ops += [wf1e, bf1e, wf2e, bf2e]
    w_specs += [
        pl.BlockSpec((HP, 4 * HP), lambda i: (0, 0)),
        pl.BlockSpec((HP, 1), lambda i: (0, 0)),
        pl.BlockSpec((8, HP), lambda i: (0, 0)),
        pl.BlockSpec((1, 8), lambda i: (0, 0)),
    ]

    nodesT = jnp.transpose(nodes, (2, 0, 1))

    grid = (B // G,)
    outG = pl.pallas_call(
        _fused_body,
        grid=grid,
        in_specs=[
            pl.BlockSpec((ATOM_DIM, G, N), lambda i: (0, i, 0)),
            pl.BlockSpec((G, N, N), lambda i: (i, 0, 0)),
            *w_specs,
        ],
        out_specs=pl.BlockSpec((G, 8), lambda i: (i, 0)),
        out_shape=jax.ShapeDtypeStruct((B, 8), jnp.float32),
        compiler_params=pltpu.CompilerParams(
            dimension_semantics=("parallel",),
        ),
    )(nodesT, adj, *w_ops)
    return outG[:, :2]
